# baseline (speedup 1.0000x reference)
import functools

import jax
import jax.numpy as jnp
from jax import lax
from jax.experimental import pallas as pl
from jax.experimental.pallas import tpu as pltpu
from jax.experimental.pallas import tpu_sc as plsc

L = 128
Q = 21
QQ = Q * Q
LL = L * L
LAMBDA_H = 0.01
LAMBDA_J = 0.01

J_CBLOCK = 512
J_NSTREAM = 4
J_STEPS = LL // (J_NSTREAM * J_CBLOCK)


def _sc_gather(jcol_flat, sigma_ri):
    mesh = plsc.VectorSubcoreMesh(
        core_axis_name="c", subcore_axis_name="s", num_cores=1)

    @functools.partial(
        pl.kernel,
        mesh=mesh,
        out_type=jax.ShapeDtypeStruct((Q, L), jnp.float32),
        scratch_types=[
            pltpu.VMEM((L,), jnp.int32),
            pltpu.VMEM((L,), jnp.int32),
            pltpu.VMEM((L,), jnp.float32),
            pltpu.SemaphoreType.DMA,
        ],
    )
    def k(jf_hbm, sig_hbm, out_hbm, sig_v, idx_v, vals_v, sem):
        sid = lax.axis_index("s")
        for rep in range(2):
            row = sid + rep * 16

            @pl.when(row < Q)
            def _():
                pltpu.sync_copy(sig_hbm.at[row], sig_v)

                for c in range(L // 16):
                    sl = pl.ds(c * 16, 16)
                    idx_v[sl] = (sig_v[sl] * L
                                 + (lax.iota(jnp.int32, 16) + c * 16))
                pltpu.async_copy(jf_hbm.at[idx_v], vals_v, sem).wait()
                pltpu.sync_copy(vals_v, out_hbm.at[row])

    return k(jcol_flat, sigma_ri)


def _ssq_body(*refs):
    (js, (out_ref, acc_ref)) = refs[:J_NSTREAM], refs[J_NSTREAM:]
    i = pl.program_id(0)

    @pl.when(i == 0)
    def _():
        acc_ref[0] = 0.0

    tot = jnp.float32(0.0)
    for j_ref in js:
        blk = j_ref[...]
        tot += jnp.sum(blk * blk)
    acc_ref[0] += tot

    @pl.when(i == J_STEPS - 1)
    def _():
        out_ref[0] = acc_ref[0]


def _tc_ssq(j2d):
    def _spec(k):
        return pl.BlockSpec((QQ, J_CBLOCK), lambda i, _k=k: (0, i + _k * J_STEPS))

    return pl.pallas_call(
        _ssq_body,
        grid=(J_STEPS,),
        in_specs=[_spec(k) for k in range(J_NSTREAM)],
        out_specs=pl.BlockSpec(memory_space=pltpu.SMEM),
        out_shape=jax.ShapeDtypeStruct((1,), jnp.float32),
        scratch_shapes=[pltpu.SMEM((1,), jnp.float32)],
    )(*([j2d] * J_NSTREAM))


def _epilogue_body(r_ref, sr_ref, wb_ref, ssq_j_ref, h_ref, vals_ref, out_ref):
    r0 = r_ref[0]
    sr = sr_ref[0]
    col = lax.broadcasted_iota(jnp.int32, (Q, L), 1)
    colmask = (col != r0).astype(jnp.float32)
    j_l = jnp.sum(vals_ref[...] * colmask, axis=1, keepdims=True)
    onehot_r = (col == r0).astype(jnp.float32)
    h_all = h_ref[...]
    h_r = jnp.sum(h_all * onehot_r, axis=1, keepdims=True)
    s = h_r + j_l
    denom = jnp.sum(jnp.exp(s))
    row = lax.broadcasted_iota(jnp.int32, (Q, 1), 0)
    pick = jnp.sum(s * (row == sr).astype(jnp.float32))
    ssq_h = jnp.sum(h_all * h_all)
    out_ref[0] = ((-pick + jnp.log(denom)) * wb_ref[0]
                  + LAMBDA_H * ssq_h + LAMBDA_J * ssq_j_ref[0])


def _tc_epilogue(r_i, sr_i, w_b, ssq_j, H_weight, vals):
    return pl.pallas_call(
        _epilogue_body,
        in_specs=[
            pl.BlockSpec(memory_space=pltpu.SMEM),
            pl.BlockSpec(memory_space=pltpu.SMEM),
            pl.BlockSpec(memory_space=pltpu.SMEM),
            pl.BlockSpec(memory_space=pltpu.SMEM),
            pl.BlockSpec((Q, L), lambda: (0, 0)),
            pl.BlockSpec((Q, L), lambda: (0, 0)),
        ],
        out_specs=pl.BlockSpec(memory_space=pltpu.SMEM),
        out_shape=jax.ShapeDtypeStruct((1,), jnp.float32),
    )(r_i, sr_i, w_b, ssq_j, H_weight, vals)


def kernel(sigma_r, sigma_i, sigma_ri, r, w_b, H_weight, J_weight):
    del sigma_i
    r_i = r.astype(jnp.int32)
    jcol = lax.dynamic_slice(J_weight, (0, r_i[0] * L), (QQ, L)).reshape(QQ * L)
    vals = _sc_gather(jcol, sigma_ri.astype(jnp.int32))
    ssq_j = _tc_ssq(J_weight)
    return _tc_epilogue(r_i, sigma_r.astype(jnp.int32), w_b, ssq_j,
                        H_weight, vals)


# speedup vs baseline: 1.0533x; 1.0533x over previous
import functools

import jax
import jax.numpy as jnp
from jax import lax
from jax.experimental import pallas as pl
from jax.experimental.pallas import tpu as pltpu
from jax.experimental.pallas import tpu_sc as plsc

L = 128
Q = 21
QQ = Q * Q
LL = L * L
LAMBDA_H = 0.01
LAMBDA_J = 0.01

J_CBLOCK = 1024
J_NSTREAM = 4
J_STEPS = LL // (J_NSTREAM * J_CBLOCK)


def _sc_gather(jcol_flat, sigma_ri):
    mesh = plsc.VectorSubcoreMesh(
        core_axis_name="c", subcore_axis_name="s", num_cores=1)

    @functools.partial(
        pl.kernel,
        mesh=mesh,
        out_type=jax.ShapeDtypeStruct((Q, L), jnp.float32),
        scratch_types=[
            pltpu.VMEM((L,), jnp.int32),
            pltpu.VMEM((L,), jnp.int32),
            pltpu.VMEM((L,), jnp.float32),
            pltpu.SemaphoreType.DMA,
        ],
    )
    def k(jf_hbm, sig_hbm, out_hbm, sig_v, idx_v, vals_v, sem):
        sid = lax.axis_index("s")
        for rep in range(2):
            row = sid + rep * 16

            @pl.when(row < Q)
            def _():
                pltpu.sync_copy(sig_hbm.at[row], sig_v)

                for c in range(L // 16):
                    sl = pl.ds(c * 16, 16)
                    idx_v[sl] = (sig_v[sl] * L
                                 + (lax.iota(jnp.int32, 16) + c * 16))
                pltpu.async_copy(jf_hbm.at[idx_v], vals_v, sem).wait()
                pltpu.sync_copy(vals_v, out_hbm.at[row])

    return k(jcol_flat, sigma_ri)


def _ssq_body(*refs):
    (js, (out_ref, acc_ref)) = refs[:J_NSTREAM], refs[J_NSTREAM:]
    i = pl.program_id(0)

    @pl.when(i == 0)
    def _():
        acc_ref[0] = 0.0

    tot = jnp.float32(0.0)
    for j_ref in js:
        blk = j_ref[...]
        tot += jnp.sum(blk * blk)
    acc_ref[0] += tot

    @pl.when(i == J_STEPS - 1)
    def _():
        out_ref[0] = acc_ref[0]


def _tc_ssq(j2d):
    def _spec(k):
        return pl.BlockSpec((QQ, J_CBLOCK), lambda i, _k=k: (0, i + _k * J_STEPS))

    return pl.pallas_call(
        _ssq_body,
        grid=(J_STEPS,),
        in_specs=[_spec(k) for k in range(J_NSTREAM)],
        out_specs=pl.BlockSpec(memory_space=pltpu.SMEM),
        out_shape=jax.ShapeDtypeStruct((1,), jnp.float32),
        scratch_shapes=[pltpu.SMEM((1,), jnp.float32)],
    )(*([j2d] * J_NSTREAM))


def _epilogue_body(r_ref, sr_ref, wb_ref, ssq_j_ref, h_ref, vals_ref, out_ref):
    r0 = r_ref[0]
    sr = sr_ref[0]
    col = lax.broadcasted_iota(jnp.int32, (Q, L), 1)
    colmask = (col != r0).astype(jnp.float32)
    j_l = jnp.sum(vals_ref[...] * colmask, axis=1, keepdims=True)
    onehot_r = (col == r0).astype(jnp.float32)
    h_all = h_ref[...]
    h_r = jnp.sum(h_all * onehot_r, axis=1, keepdims=True)
    s = h_r + j_l
    denom = jnp.sum(jnp.exp(s))
    row = lax.broadcasted_iota(jnp.int32, (Q, 1), 0)
    pick = jnp.sum(s * (row == sr).astype(jnp.float32))
    ssq_h = jnp.sum(h_all * h_all)
    out_ref[0] = ((-pick + jnp.log(denom)) * wb_ref[0]
                  + LAMBDA_H * ssq_h + LAMBDA_J * ssq_j_ref[0])


def _tc_epilogue(r_i, sr_i, w_b, ssq_j, H_weight, vals):
    return pl.pallas_call(
        _epilogue_body,
        in_specs=[
            pl.BlockSpec(memory_space=pltpu.SMEM),
            pl.BlockSpec(memory_space=pltpu.SMEM),
            pl.BlockSpec(memory_space=pltpu.SMEM),
            pl.BlockSpec(memory_space=pltpu.SMEM),
            pl.BlockSpec((Q, L), lambda: (0, 0)),
            pl.BlockSpec((Q, L), lambda: (0, 0)),
        ],
        out_specs=pl.BlockSpec(memory_space=pltpu.SMEM),
        out_shape=jax.ShapeDtypeStruct((1,), jnp.float32),
    )(r_i, sr_i, w_b, ssq_j, H_weight, vals)


def kernel(sigma_r, sigma_i, sigma_ri, r, w_b, H_weight, J_weight):
    del sigma_i
    r_i = r.astype(jnp.int32)
    jcol = lax.dynamic_slice(J_weight, (0, r_i[0] * L), (QQ, L)).reshape(QQ * L)
    vals = _sc_gather(jcol, sigma_ri.astype(jnp.int32))
    ssq_j = _tc_ssq(J_weight)
    return _tc_epilogue(r_i, sigma_r.astype(jnp.int32), w_b, ssq_j,
                        H_weight, vals)


# speedup vs baseline: 1.0536x; 1.0003x over previous
import functools

import jax
import jax.numpy as jnp
from jax import lax
from jax.experimental import pallas as pl
from jax.experimental.pallas import tpu as pltpu
from jax.experimental.pallas import tpu_sc as plsc

L = 128
Q = 21
QQ = Q * Q
LL = L * L
LAMBDA_H = 0.01
LAMBDA_J = 0.01

J_CBLOCK = 1024
J_NSTREAM = 4
J_STEPS = LL // (J_NSTREAM * J_CBLOCK)


def _sc_gather(jcol_flat, sigma_ri):
    mesh = plsc.VectorSubcoreMesh(
        core_axis_name="c", subcore_axis_name="s", num_cores=1)

    @functools.partial(
        pl.kernel,
        mesh=mesh,
        out_type=jax.ShapeDtypeStruct((Q, L), jnp.float32),
        scratch_types=[
            pltpu.VMEM((L,), jnp.int32),
            pltpu.VMEM((L,), jnp.int32),
            pltpu.VMEM((L,), jnp.int32),
            pltpu.VMEM((L,), jnp.int32),
            pltpu.VMEM((L,), jnp.float32),
            pltpu.VMEM((L,), jnp.float32),
            pltpu.SemaphoreType.DMA,
            pltpu.SemaphoreType.DMA,
            pltpu.SemaphoreType.DMA,
            pltpu.SemaphoreType.DMA,
        ],
    )
    def k(jf_hbm, sig_hbm, out_hbm, sig0, sig1, idx0, idx1, val0, val1,
          sa, sb, ga, gb):
        sid = lax.axis_index("s")
        row1 = sid + 16
        rep2 = row1 < Q

        pltpu.async_copy(sig_hbm.at[sid], sig0, sa)

        @pl.when(rep2)
        def _():
            pltpu.async_copy(sig_hbm.at[row1], sig1, sb)

        pltpu.make_async_copy(sig_hbm.at[sid], sig0, sa).wait()

        @pl.when(rep2)
        def _():
            pltpu.make_async_copy(sig_hbm.at[row1], sig1, sb).wait()

        for c in range(L // 16):
            sl = pl.ds(c * 16, 16)
            lanes = lax.iota(jnp.int32, 16) + c * 16
            idx0[sl] = sig0[sl] * L + lanes

        @pl.when(rep2)
        def _():
            for c in range(L // 16):
                sl = pl.ds(c * 16, 16)
                lanes = lax.iota(jnp.int32, 16) + c * 16
                idx1[sl] = sig1[sl] * L + lanes

        pltpu.async_copy(jf_hbm.at[idx0], val0, ga)

        @pl.when(rep2)
        def _():
            pltpu.async_copy(jf_hbm.at[idx1], val1, gb)

        pltpu.make_async_copy(jf_hbm.at[idx0], val0, ga).wait()

        @pl.when(rep2)
        def _():
            pltpu.make_async_copy(jf_hbm.at[idx1], val1, gb).wait()

        pltpu.async_copy(val0, out_hbm.at[sid], sa)

        @pl.when(rep2)
        def _():
            pltpu.async_copy(val1, out_hbm.at[row1], sb)

        pltpu.make_async_copy(val0, out_hbm.at[sid], sa).wait()

        @pl.when(rep2)
        def _():
            pltpu.make_async_copy(val1, out_hbm.at[row1], sb).wait()

    return k(jcol_flat, sigma_ri)


def _ssq_body(*refs):
    (js, (out_ref, acc_ref)) = refs[:J_NSTREAM], refs[J_NSTREAM:]
    i = pl.program_id(0)

    @pl.when(i == 0)
    def _():
        acc_ref[0] = 0.0

    tot = jnp.float32(0.0)
    for j_ref in js:
        blk = j_ref[...]
        tot += jnp.sum(blk * blk)
    acc_ref[0] += tot

    @pl.when(i == J_STEPS - 1)
    def _():
        out_ref[0] = acc_ref[0]


def _tc_ssq(j2d):
    def _spec(k):
        return pl.BlockSpec((QQ, J_CBLOCK), lambda i, _k=k: (0, i + _k * J_STEPS))

    return pl.pallas_call(
        _ssq_body,
        grid=(J_STEPS,),
        in_specs=[_spec(k) for k in range(J_NSTREAM)],
        out_specs=pl.BlockSpec(memory_space=pltpu.SMEM),
        out_shape=jax.ShapeDtypeStruct((1,), jnp.float32),
        scratch_shapes=[pltpu.SMEM((1,), jnp.float32)],
    )(*([j2d] * J_NSTREAM))


def _epilogue_body(r_ref, sr_ref, wb_ref, ssq_j_ref, h_ref, vals_ref, out_ref):
    r0 = r_ref[0]
    sr = sr_ref[0]
    col = lax.broadcasted_iota(jnp.int32, (Q, L), 1)
    colmask = (col != r0).astype(jnp.float32)
    j_l = jnp.sum(vals_ref[...] * colmask, axis=1, keepdims=True)
    onehot_r = (col == r0).astype(jnp.float32)
    h_all = h_ref[...]
    h_r = jnp.sum(h_all * onehot_r, axis=1, keepdims=True)
    s = h_r + j_l
    denom = jnp.sum(jnp.exp(s))
    row = lax.broadcasted_iota(jnp.int32, (Q, 1), 0)
    pick = jnp.sum(s * (row == sr).astype(jnp.float32))
    ssq_h = jnp.sum(h_all * h_all)
    out_ref[0] = ((-pick + jnp.log(denom)) * wb_ref[0]
                  + LAMBDA_H * ssq_h + LAMBDA_J * ssq_j_ref[0])


def _tc_epilogue(r_i, sr_i, w_b, ssq_j, H_weight, vals):
    return pl.pallas_call(
        _epilogue_body,
        in_specs=[
            pl.BlockSpec(memory_space=pltpu.SMEM),
            pl.BlockSpec(memory_space=pltpu.SMEM),
            pl.BlockSpec(memory_space=pltpu.SMEM),
            pl.BlockSpec(memory_space=pltpu.SMEM),
            pl.BlockSpec((Q, L), lambda: (0, 0)),
            pl.BlockSpec((Q, L), lambda: (0, 0)),
        ],
        out_specs=pl.BlockSpec(memory_space=pltpu.SMEM),
        out_shape=jax.ShapeDtypeStruct((1,), jnp.float32),
    )(r_i, sr_i, w_b, ssq_j, H_weight, vals)


def kernel(sigma_r, sigma_i, sigma_ri, r, w_b, H_weight, J_weight):
    del sigma_i
    r_i = r.astype(jnp.int32)
    jcol = lax.dynamic_slice(J_weight, (0, r_i[0] * L), (QQ, L)).reshape(QQ * L)
    vals = _sc_gather(jcol, sigma_ri.astype(jnp.int32))
    ssq_j = _tc_ssq(J_weight)
    return _tc_epilogue(r_i, sigma_r.astype(jnp.int32), w_b, ssq_j,
                        H_weight, vals)
